# Initial kernel scaffold; baseline (speedup 1.0000x reference)
import jax
import jax.numpy as jnp
from jax.experimental import pallas as pl


def kernel(confidences, accuracies):
    raise NotImplementedError("write your pallas kernel here")


# speedup vs baseline: 1.8027x; 1.8027x over previous
import jax
import jax.numpy as jnp
from jax.experimental import pallas as pl
from jax.experimental.pallas import tpu as pltpu

N_BINS = 15
LANES = 128
BLOCK_ROWS = 8192


def _ece_body(c_ref, a_ref, o_ref, acc_ref):
    step = pl.program_id(0)

    @pl.when(step == 0)
    def _init():
        acc_ref[...] = jnp.zeros_like(acc_ref)

    c = c_ref[...]
    a = a_ref[...]
    b = jnp.ceil(c * jnp.float32(N_BINS)) - jnp.float32(1.0)
    ones = jnp.ones_like(c)
    zeros = jnp.zeros_like(c)
    for k in range(N_BINS):
        m = b == jnp.float32(k)
        acc_ref[k, :] += jnp.sum(jnp.where(m, ones, zeros), axis=0)
        acc_ref[k + 16, :] += jnp.sum(jnp.where(m, c, zeros), axis=0)
        acc_ref[k + 32, :] += jnp.sum(jnp.where(m, a, zeros), axis=0)

    @pl.when(step == pl.num_programs(0) - 1)
    def _finish():
        tot = jnp.sum(acc_ref[...], axis=1, keepdims=True)
        cnt = tot[0:N_BINS, :]
        csum = tot[16:16 + N_BINS, :]
        asum = tot[32:32 + N_BINS, :]
        n_total = jnp.float32(pl.num_programs(0) * BLOCK_ROWS * LANES)
        safe = jnp.maximum(cnt, 1.0)
        diff = (csum - asum) / safe
        contrib = diff * diff * (cnt / n_total)
        contrib = jnp.where(cnt > 0, contrib, 0.0)
        o_ref[...] = jnp.sum(contrib, axis=(0, 1), keepdims=True)


def kernel(confidences, accuracies):
    n = confidences.shape[0]
    rows = n // LANES
    c2 = confidences.reshape(rows, LANES)
    a2 = accuracies.reshape(rows, LANES)
    grid = rows // BLOCK_ROWS
    out = pl.pallas_call(
        _ece_body,
        grid=(grid,),
        in_specs=[
            pl.BlockSpec((BLOCK_ROWS, LANES), lambda i: (i, 0)),
            pl.BlockSpec((BLOCK_ROWS, LANES), lambda i: (i, 0)),
        ],
        out_specs=pl.BlockSpec((1, 1), lambda i: (0, 0)),
        out_shape=jax.ShapeDtypeStruct((1, 1), jnp.float32),
        scratch_shapes=[pltpu.VMEM((48, LANES), jnp.float32)],
    )(c2, a2)
    return out[0, 0]
